# baseline (speedup 1.0000x reference)
import jax
import jax.numpy as jnp
from jax import lax
from jax.experimental import pallas as pl
from jax.experimental.pallas import tpu as pltpu
from jax.experimental.pallas import tpu_sc as plsc

B = 16384
F = 26
V = 100000
D = 16

NC = 2
NS = 16
NW = NC * NS
BW = B // NW
CB = 128
NCHUNK = BW // CB
L = 16


def _sc_body(idx_hbm, dnn_hbm, lin_hbm, out_hbm,
             idx_v, rows_v, lin_v, out_v, sem_i, sem_d, sem_l):
    wid = lax.axis_index("s") * NC + lax.axis_index("c")
    base = wid * BW

    icops = [
        pltpu.async_copy(idx_hbm.at[pl.ds(f * B + base, BW)],
                         idx_v.at[pl.ds(f * BW, BW)], sem_i)
        for f in range(F)
    ]
    for cp in icops:
        cp.wait()

    lane = lax.broadcasted_iota(jnp.int32, (L,), 0)

    for c in range(NCHUNK):
        cbase = c * CB
        dcops = []
        lcops = []
        for f in range(F):
            isl = idx_v.at[pl.ds(f * BW + cbase, CB)]
            dcops.append(pltpu.async_copy(
                dnn_hbm.at[isl], rows_v.at[pl.ds(f * CB, CB)], sem_d))
            lcops.append(pltpu.async_copy(
                lin_hbm.at[isl], lin_v.at[pl.ds(f * CB, CB)], sem_l))
        for cp in dcops:
            cp.wait()
        for cp in lcops:
            cp.wait()

        def g_body(g, carry):
            b0 = g * L
            lacc = jnp.zeros((L,), jnp.float32)
            for f in range(F):
                lacc = lacc + lin_v[pl.ds(f * CB + b0, L)]
            row_base = b0 + lane
            fm = jnp.zeros((L,), jnp.float32)
            for d in range(D):
                idx_d = jnp.full((L,), d, jnp.int32)
                s = jnp.zeros((L,), jnp.float32)
                q = jnp.zeros((L,), jnp.float32)
                for f in range(F):
                    v = plsc.load_gather(rows_v, [row_base + f * CB, idx_d])
                    s = s + v
                    q = q + v * v
                fm = fm + (s * s - q)
            logit = lacc + 0.5 * fm
            pred = 1.0 / (1.0 + jnp.exp(-logit))
            out_v[pl.ds(b0, L)] = pred
            return carry

        lax.fori_loop(0, CB // L, g_body, 0)
        pltpu.sync_copy(out_v, out_hbm.at[pl.ds(base + cbase, CB)])


def kernel(X, linear_tables, dnn_tables):
    idx = X.astype(jnp.int32) + (jnp.arange(F, dtype=jnp.int32) * V)[None, :]
    idx_flat = idx.T.reshape(F * B)
    dnn2d = dnn_tables.reshape(F * V, D)
    lin1d = linear_tables.reshape(F * V)

    mesh = plsc.VectorSubcoreMesh(core_axis_name="c", subcore_axis_name="s",
                                  num_cores=NC, num_subcores=NS)
    run = pl.kernel(
        _sc_body,
        out_type=jax.ShapeDtypeStruct((B,), jnp.float32),
        mesh=mesh,
        compiler_params=pltpu.CompilerParams(
            needs_layout_passes=False, use_tc_tiling_on_sc=False),
        scratch_types=[
            pltpu.VMEM((F * BW,), jnp.int32),
            pltpu.VMEM((F * CB, D), jnp.float32),
            pltpu.VMEM((F * CB,), jnp.float32),
            pltpu.VMEM((CB,), jnp.float32),
            pltpu.SemaphoreType.DMA,
            pltpu.SemaphoreType.DMA,
            pltpu.SemaphoreType.DMA,
        ],
    )
    pred = run(idx_flat, dnn2d, lin1d)
    return pred.reshape(B, 1)


# speedup vs baseline: 1.0309x; 1.0309x over previous
import jax
import jax.numpy as jnp
from jax import lax
from jax.experimental import pallas as pl
from jax.experimental.pallas import tpu as pltpu
from jax.experimental.pallas import tpu_sc as plsc

B = 16384
F = 26
V = 100000
D = 16

NC = 2
NS = 16
NW = NC * NS
BW = B // NW
CB = 128
NCHUNK = BW // CB
L = 16


def _sc_body(idx_hbm, dnn_hbm, lin_hbm, out_hbm,
             idx_v, rows_v, lin_v, out_v, stage_s, stage_q,
             sem_i, sem_d0, sem_d1):
    wid = lax.axis_index("s") * NC + lax.axis_index("c")
    base = wid * BW

    icops = [
        pltpu.async_copy(idx_hbm.at[pl.ds(f * B + base, BW)],
                         idx_v.at[pl.ds(f * BW, BW)], sem_i)
        for f in range(F)
    ]
    for cp in icops:
        cp.wait()

    lane = lax.broadcasted_iota(jnp.int32, (L,), 0)
    sems = (sem_d0, sem_d1)

    def fire(c):
        cbase = c * CB
        buf = c % 2
        sem = sems[buf]
        cops = []
        for f in range(F):
            isl = idx_v.at[pl.ds(f * BW + cbase, CB)]
            cops.append(pltpu.async_copy(
                dnn_hbm.at[isl], rows_v.at[pl.ds((buf * F + f) * CB, CB)],
                sem))
            cops.append(pltpu.async_copy(
                lin_hbm.at[isl], lin_v.at[pl.ds((buf * F + f) * CB, CB)],
                sem))
        return cops

    pending = fire(0)
    for c in range(NCHUNK):
        nxt = fire(c + 1) if c + 1 < NCHUNK else []
        for cp in pending:
            cp.wait()
        pending = nxt
        buf = c % 2

        def g_body(g, carry):
            b0 = g * L
            lacc = jnp.zeros((L,), jnp.float32)
            for f in range(F):
                lacc = lacc + lin_v[pl.ds((buf * F + f) * CB + b0, L)]
            lane16 = lane * L
            for j in range(L):
                r0 = buf * (F * CB) + b0 + j
                s = jnp.zeros((L,), jnp.float32)
                q = jnp.zeros((L,), jnp.float32)
                for f in range(F):
                    v = rows_v[r0 + f * CB, :]
                    s = s + v
                    q = q + v * v
                idx_st = lane16 + j
                plsc.store_scatter(stage_s, [idx_st], s)
                plsc.store_scatter(stage_q, [idx_st], q)
            fm = jnp.zeros((L,), jnp.float32)
            for d in range(D):
                sc = stage_s[pl.ds(d * L, L)]
                qc = stage_q[pl.ds(d * L, L)]
                fm = fm + (sc * sc - qc)
            logit = lacc + 0.5 * fm
            pred = 1.0 / (1.0 + jnp.exp(-logit))
            out_v[pl.ds(b0, L)] = pred
            return carry

        lax.fori_loop(0, CB // L, g_body, 0)
        pltpu.sync_copy(out_v, out_hbm.at[pl.ds(base + c * CB, CB)])


def kernel(X, linear_tables, dnn_tables):
    idx = X.astype(jnp.int32) + (jnp.arange(F, dtype=jnp.int32) * V)[None, :]
    idx_flat = idx.T.reshape(F * B)
    dnn2d = dnn_tables.reshape(F * V, D)
    lin1d = linear_tables.reshape(F * V)

    mesh = plsc.VectorSubcoreMesh(core_axis_name="c", subcore_axis_name="s",
                                  num_cores=NC, num_subcores=NS)
    run = pl.kernel(
        _sc_body,
        out_type=jax.ShapeDtypeStruct((B,), jnp.float32),
        mesh=mesh,
        compiler_params=pltpu.CompilerParams(
            needs_layout_passes=False, use_tc_tiling_on_sc=False),
        scratch_types=[
            pltpu.VMEM((F * BW,), jnp.int32),
            pltpu.VMEM((2 * F * CB, D), jnp.float32),
            pltpu.VMEM((2 * F * CB,), jnp.float32),
            pltpu.VMEM((CB,), jnp.float32),
            pltpu.VMEM((D * L,), jnp.float32),
            pltpu.VMEM((D * L,), jnp.float32),
            pltpu.SemaphoreType.DMA,
            pltpu.SemaphoreType.DMA,
            pltpu.SemaphoreType.DMA,
        ],
    )
    pred = run(idx_flat, dnn2d, lin1d)
    return pred.reshape(B, 1)


# speedup vs baseline: 1.7510x; 1.6986x over previous
import jax
import jax.numpy as jnp
from jax import lax
from jax.experimental import pallas as pl
from jax.experimental.pallas import tpu as pltpu
from jax.experimental.pallas import tpu_sc as plsc

B = 16384
F = 26
V = 100000
D = 16
P = D + 1

NC = 2
NS = 16
NW = NC * NS
BW = B // NW
CB = 128
NCHUNK = BW // CB
L = 16


def _sc_body(idx_hbm, dnn_hbm, lin_hbm, out_hbm,
             idx_v, pl_v, out_v, sem_i, sem_d0, sem_d1):
    wid = lax.axis_index("s") * NC + lax.axis_index("c")
    base = wid * BW

    icops = [
        pltpu.async_copy(idx_hbm.at[pl.ds(f * B + base, BW)],
                         idx_v.at[pl.ds(f * BW, BW)], sem_i)
        for f in range(F)
    ]
    for cp in icops:
        cp.wait()

    sems = (sem_d0, sem_d1)

    def fire(c):
        buf = c % 2
        sem = sems[buf]

        def fire_f(f, carry):
            isl = idx_v.at[pl.ds(f * BW + c * CB, CB)]
            fb = (buf * F + f) * P
            pltpu.async_copy(lin_hbm.at[pl.ds(f * V, V)].at[isl],
                             pl_v.at[pl.ds(fb * CB, CB)], sem)
            for d in range(D):
                pltpu.async_copy(dnn_hbm.at[pl.ds((f * D + d) * V, V)].at[isl],
                                 pl_v.at[pl.ds((fb + 1 + d) * CB, CB)], sem)
            return carry

        lax.fori_loop(0, F, fire_f, 0)

    def drain(c):
        buf = c % 2
        pltpu.make_async_copy(
            dnn_hbm.at[pl.ds(0, F * P * CB)],
            pl_v.at[pl.ds(buf * F * P * CB, F * P * CB)],
            sems[buf]).wait()

    fire(0)
    for c in range(NCHUNK):
        if c + 1 < NCHUNK:
            fire(c + 1)
        drain(c)
        buf = c % 2

        def g_body(g, carry):
            b0 = g * L
            lacc = jnp.zeros((L,), jnp.float32)
            for f in range(F):
                lacc = lacc + pl_v[pl.ds(((buf * F + f) * P) * CB + b0, L)]
            fm = jnp.zeros((L,), jnp.float32)
            for d in range(D):
                s = jnp.zeros((L,), jnp.float32)
                q = jnp.zeros((L,), jnp.float32)
                for f in range(F):
                    v = pl_v[pl.ds(((buf * F + f) * P + 1 + d) * CB + b0, L)]
                    s = s + v
                    q = q + v * v
                fm = fm + (s * s - q)
            logit = lacc + 0.5 * fm
            pred = 1.0 / (1.0 + jnp.exp(-logit))
            out_v[pl.ds(b0, L)] = pred
            return carry

        lax.fori_loop(0, CB // L, g_body, 0)
        pltpu.sync_copy(out_v, out_hbm.at[pl.ds(base + c * CB, CB)])


def kernel(X, linear_tables, dnn_tables):
    idx_flat = X.astype(jnp.int32).T.reshape(F * B)
    dnn_flat = jnp.transpose(dnn_tables, (0, 2, 1)).reshape(F * D * V)
    lin_flat = jnp.transpose(linear_tables, (0, 2, 1)).reshape(F * V)

    mesh = plsc.VectorSubcoreMesh(core_axis_name="c", subcore_axis_name="s",
                                  num_cores=NC, num_subcores=NS)
    run = pl.kernel(
        _sc_body,
        out_type=jax.ShapeDtypeStruct((B,), jnp.float32),
        mesh=mesh,
        compiler_params=pltpu.CompilerParams(
            needs_layout_passes=False, use_tc_tiling_on_sc=False),
        scratch_types=[
            pltpu.VMEM((F * BW,), jnp.int32),
            pltpu.VMEM((2 * F * P * CB,), jnp.float32),
            pltpu.VMEM((CB,), jnp.float32),
            pltpu.SemaphoreType.DMA,
            pltpu.SemaphoreType.DMA,
            pltpu.SemaphoreType.DMA,
        ],
    )
    pred = run(idx_flat, dnn_flat, lin_flat)
    return pred.reshape(B, 1)
